# Initial kernel scaffold; baseline (speedup 1.0000x reference)
import jax
import jax.numpy as jnp
from jax.experimental import pallas as pl


def kernel(x, edge_index, W1l, b1, W1r, W2l, b2, W2r):
    raise NotImplementedError("write your pallas kernel here")


# speedup vs baseline: 5.5831x; 5.5831x over previous
import functools

import jax
import jax.numpy as jnp
from jax import lax
from jax.experimental import pallas as pl
from jax.experimental.pallas import tpu as pltpu
from jax.experimental.pallas import tpu_sc as plsc

_NC = 1
_NS = 16
_CHUNK = 50
_GPC = 40


@functools.lru_cache(maxsize=None)
def _sc_segment_sum(npad, e, d):
  nw = _NC * _NS
  assert e % (nw * _CHUNK * _GPC) == 0 and npad % (_NS * 8) == 0
  nchunks = e // (nw * _CHUNK)
  ngen = nchunks // _GPC
  rpt = npad // _NS
  mesh = plsc.VectorSubcoreMesh(core_axis_name="c", subcore_axis_name="s",
                                num_cores=_NC, num_subcores=_NS)

  def body(h_hbm, src_hbm, dst_hbm, znd_hbm, p_out,
           srcv, dstv, buf0, buf1, g0, g1, s0, s1, acc):
    c = lax.axis_index("c")
    s = lax.axis_index("s")
    row0 = (c * _NS + s) * nchunks
    r0 = s * rpt

    pltpu.sync_copy(znd_hbm.at[pl.ds(r0, rpt)], acc.at[pl.ds(r0, rpt)])
    plsc.subcore_barrier()

    def gather(j, buf, sem):
      pltpu.async_copy(h_hbm.at[srcv.at[j]], buf, sem)

    def wait_gather(j, buf, sem):
      pltpu.make_async_copy(h_hbm.at[srcv.at[j]], buf, sem).wait()

    def scat(j, buf, sem):
      pltpu.async_copy(buf, acc.at[dstv.at[j]], sem, add=True)

    def wait_scat(j, buf, sem):
      pltpu.make_async_copy(buf, acc.at[dstv.at[j]], sem).wait()

    def gen_body(g, carry):
      base = row0 + g * _GPC
      pltpu.sync_copy(src_hbm.at[pl.ds(base, _GPC)], srcv)
      pltpu.sync_copy(dst_hbm.at[pl.ds(base, _GPC)], dstv)

      gather(0, buf0, g0)
      gather(1, buf1, g1)
      wait_gather(0, buf0, g0)
      scat(0, buf0, s0)

      def step(i, cy):
        a = 2 * i
        wait_scat(a, buf0, s0)
        gather(a + 2, buf0, g0)
        wait_gather(a + 1, buf1, g1)
        scat(a + 1, buf1, s1)
        wait_scat(a + 1, buf1, s1)
        gather(a + 3, buf1, g1)
        wait_gather(a + 2, buf0, g0)
        scat(a + 2, buf0, s0)
        return cy

      lax.fori_loop(0, (_GPC - 4) // 2 + 1, step, 0)
      last = _GPC - 1
      wait_gather(last, buf1, g1)
      scat(last, buf1, s1)
      wait_scat(last - 1, buf0, s0)
      wait_scat(last, buf1, s1)
      return carry

    lax.fori_loop(0, ngen, gen_body, 0)

    plsc.subcore_barrier()
    pltpu.sync_copy(acc.at[pl.ds(r0, rpt)], p_out.at[pl.ds(r0, rpt)])

  return pl.kernel(
      body,
      out_type=[jax.ShapeDtypeStruct((npad, d), jnp.float32)],
      mesh=mesh,
      scratch_types=[
          pltpu.VMEM((_GPC, _CHUNK), jnp.int32),
          pltpu.VMEM((_GPC, _CHUNK), jnp.int32),
          pltpu.VMEM((_CHUNK, d), jnp.float32),
          pltpu.VMEM((_CHUNK, d), jnp.float32),
          pltpu.SemaphoreType.DMA,
          pltpu.SemaphoreType.DMA,
          pltpu.SemaphoreType.DMA,
          pltpu.SemaphoreType.DMA,
          pltpu.VMEM_SHARED((npad, d), jnp.float32),
      ])


@functools.lru_cache(maxsize=None)
def _sc_count(npad, e):
  nw = _NC * _NS
  epw = e // nw
  assert e % (nw * 16) == 0 and npad % 8 == 0 and epw % 8 == 0
  mesh = plsc.VectorSubcoreMesh(core_axis_name="c", subcore_axis_name="s",
                                num_cores=_NC, num_subcores=_NS)

  def body(dst_hbm, cnt_out, dstv, tab):
    c = lax.axis_index("c")
    s = lax.axis_index("s")
    w = c * _NS + s

    pltpu.sync_copy(dst_hbm.at[pl.ds(w * epw, epw)], dstv)
    zeros = jnp.zeros((16,), jnp.float32)

    def zstep(i, cy):
      tab[pl.ds(i * 16, 16)] = zeros
      return cy

    lax.fori_loop(0, npad // 16, zstep, 0)
    ones = jnp.ones((16,), jnp.float32)

    def step(i, cy):
      v = dstv[pl.ds(i * 16, 16)]
      plsc.addupdate_scatter(tab, [v], ones)
      return cy

    lax.fori_loop(0, epw // 16, step, 0)
    pltpu.sync_copy(tab, cnt_out.at[pl.ds(w * npad, npad)])

  return pl.kernel(
      body,
      out_type=[jax.ShapeDtypeStruct((nw * npad,), jnp.float32)],
      mesh=mesh,
      compiler_params=pltpu.CompilerParams(needs_layout_passes=False),
      scratch_types=[
          pltpu.VMEM((epw,), jnp.int32),
          pltpu.VMEM((npad,), jnp.float32),
      ])


@functools.lru_cache(maxsize=None)
def _tc_combine(n, d, blk, relu):
  assert n % blk == 0
  nw = _NC * _NS

  def body(p_ref, c_ref, h_ref, wl_ref, b_ref, wr_ref, o_ref):
    cnt = jnp.sum(c_ref[...], axis=1)[:, None]
    mean = p_ref[...] / jnp.maximum(cnt, 1.0)
    out = (jnp.dot(mean, wl_ref[...], preferred_element_type=jnp.float32)
           + jnp.dot(h_ref[...], wr_ref[...],
                     preferred_element_type=jnp.float32)
           + b_ref[...])
    if relu:
      out = jnp.maximum(out, 0.0)
    o_ref[...] = out

  return pl.pallas_call(
      body,
      grid=(n // blk,),
      in_specs=[
          pl.BlockSpec((blk, d), lambda i: (i, 0)),
          pl.BlockSpec((blk, nw), lambda i: (i, 0)),
          pl.BlockSpec((blk, d), lambda i: (i, 0)),
          pl.BlockSpec((d, d), lambda i: (0, 0)),
          pl.BlockSpec((1, d), lambda i: (0, 0)),
          pl.BlockSpec((d, d), lambda i: (0, 0)),
      ],
      out_specs=pl.BlockSpec((blk, d), lambda i: (i, 0)),
      out_shape=jax.ShapeDtypeStruct((n, d), jnp.float32),
  )


def kernel(x, edge_index, W1l, b1, W1r, W2l, b2, W2r):
  n, d = x.shape
  e = edge_index.shape[1]
  npad = ((n + _NS * 8 - 1) // (_NS * 8)) * (_NS * 8)

  src2 = edge_index[0].reshape(e // _CHUNK, _CHUNK)
  dst2 = edge_index[1].reshape(e // _CHUNK, _CHUNK)
  znd = jnp.zeros((npad, d), jnp.float32)
  b1r = b1.reshape(1, d)
  b2r = b2.reshape(1, d)

  (c1f,) = _sc_count(npad, e)(edge_index[1])
  c1 = c1f.reshape(_NC * _NS, npad).T
  (p1,) = _sc_segment_sum(npad, e, d)(x, src2, dst2, znd)
  h = _tc_combine(n, d, 1000, True)(p1, c1, x, W1l, b1r, W1r)
  (p2,) = _sc_segment_sum(npad, e, d)(h, src2, dst2, znd)
  out = _tc_combine(n, d, 1000, False)(p2, c1, h, W2l, b2r, W2r)
  return out


# speedup vs baseline: 7.3008x; 1.3076x over previous
import functools

import jax
import jax.numpy as jnp
from jax import lax
from jax.experimental import pallas as pl
from jax.experimental.pallas import tpu as pltpu
from jax.experimental.pallas import tpu_sc as plsc

_NC = 1
_NS = 16
_CHUNK = 50
_GPC = 40


@functools.lru_cache(maxsize=None)
def _sc_segment_sum(npad, e, d):
  nw = _NC * _NS
  assert e % (nw * _CHUNK * _GPC) == 0 and npad % (_NS * 8) == 0
  nchunks = e // (nw * _CHUNK)
  ngen = nchunks // _GPC
  rpt = npad // _NS
  mesh = plsc.VectorSubcoreMesh(core_axis_name="c", subcore_axis_name="s",
                                num_cores=_NC, num_subcores=_NS)

  def body(h_hbm, src_hbm, dst_hbm, znd_hbm, p_out,
           srcv, dstv, b0, b1, b2, b3, g0, g1, g2, g3, s0, s1, s2, s3,
           acc):
    bufs = (b0, b1, b2, b3)
    gsems = (g0, g1, g2, g3)
    ssems = (s0, s1, s2, s3)
    c = lax.axis_index("c")
    s = lax.axis_index("s")
    row0 = (c * _NS + s) * nchunks
    r0 = s * rpt

    pltpu.sync_copy(znd_hbm.at[pl.ds(r0, rpt)], acc.at[pl.ds(r0, rpt)])
    plsc.subcore_barrier()

    def gather(j, k):
      pltpu.async_copy(h_hbm.at[srcv.at[j]], bufs[k], gsems[k])

    def wait_gather(j, k):
      pltpu.make_async_copy(h_hbm.at[srcv.at[j]], bufs[k],
                            gsems[k]).wait()

    def scat(j, k):
      pltpu.async_copy(bufs[k], acc.at[dstv.at[j]], ssems[k], add=True)

    def wait_scat(j, k):
      pltpu.make_async_copy(bufs[k], acc.at[dstv.at[j]], ssems[k]).wait()

    def gen_body(g, carry):
      base = row0 + g * _GPC
      pltpu.sync_copy(src_hbm.at[pl.ds(base, _GPC)], srcv)
      pltpu.sync_copy(dst_hbm.at[pl.ds(base, _GPC)], dstv)

      gather(0, 0)
      gather(1, 1)
      wait_gather(0, 0)
      scat(0, 0)
      gather(2, 2)
      wait_gather(1, 1)
      scat(1, 1)
      gather(3, 3)

      def step(i, cy):
        k0 = 4 * i + 4
        for j in range(4):
          k = k0 + j
          wait_scat(k - 4, j)
          gather(k, j)
          wait_gather(k - 2, (j + 2) % 4)
          scat(k - 2, (j + 2) % 4)
        return cy

      lax.fori_loop(0, (_GPC - 4) // 4, step, 0)
      e1, e2 = _GPC - 2, _GPC - 1
      wait_gather(e1, e1 % 4)
      scat(e1, e1 % 4)
      wait_gather(e2, e2 % 4)
      scat(e2, e2 % 4)
      for j in range(4):
        wait_scat(_GPC - 4 + j, (_GPC - 4 + j) % 4)
      return carry

    lax.fori_loop(0, ngen, gen_body, 0)

    plsc.subcore_barrier()
    pltpu.sync_copy(acc.at[pl.ds(r0, rpt)], p_out.at[pl.ds(r0, rpt)])

  return pl.kernel(
      body,
      out_type=[jax.ShapeDtypeStruct((npad, d), jnp.float32)],
      mesh=mesh,
      scratch_types=[
          pltpu.VMEM((_GPC, _CHUNK), jnp.int32),
          pltpu.VMEM((_GPC, _CHUNK), jnp.int32),
          pltpu.VMEM((_CHUNK, d), jnp.float32),
          pltpu.VMEM((_CHUNK, d), jnp.float32),
          pltpu.VMEM((_CHUNK, d), jnp.float32),
          pltpu.VMEM((_CHUNK, d), jnp.float32),
          pltpu.SemaphoreType.DMA,
          pltpu.SemaphoreType.DMA,
          pltpu.SemaphoreType.DMA,
          pltpu.SemaphoreType.DMA,
          pltpu.SemaphoreType.DMA,
          pltpu.SemaphoreType.DMA,
          pltpu.SemaphoreType.DMA,
          pltpu.SemaphoreType.DMA,
          pltpu.VMEM_SHARED((npad, d), jnp.float32),
      ])


@functools.lru_cache(maxsize=None)
def _sc_count(npad, e):
  nw = _NC * _NS
  epw = e // nw
  assert e % (nw * 16) == 0 and npad % 8 == 0 and epw % 8 == 0
  mesh = plsc.VectorSubcoreMesh(core_axis_name="c", subcore_axis_name="s",
                                num_cores=_NC, num_subcores=_NS)

  def body(dst_hbm, cnt_out, dstv, tab):
    c = lax.axis_index("c")
    s = lax.axis_index("s")
    w = c * _NS + s

    pltpu.sync_copy(dst_hbm.at[pl.ds(w * epw, epw)], dstv)
    zeros = jnp.zeros((16,), jnp.float32)

    def zstep(i, cy):
      tab[pl.ds(i * 16, 16)] = zeros
      return cy

    lax.fori_loop(0, npad // 16, zstep, 0)
    ones = jnp.ones((16,), jnp.float32)

    def step(i, cy):
      v = dstv[pl.ds(i * 16, 16)]
      plsc.addupdate_scatter(tab, [v], ones)
      return cy

    lax.fori_loop(0, epw // 16, step, 0)
    pltpu.sync_copy(tab, cnt_out.at[pl.ds(w * npad, npad)])

  return pl.kernel(
      body,
      out_type=[jax.ShapeDtypeStruct((nw * npad,), jnp.float32)],
      mesh=mesh,
      compiler_params=pltpu.CompilerParams(needs_layout_passes=False),
      scratch_types=[
          pltpu.VMEM((epw,), jnp.int32),
          pltpu.VMEM((npad,), jnp.float32),
      ])


@functools.lru_cache(maxsize=None)
def _tc_combine(n, d, blk, relu):
  assert n % blk == 0
  nw = _NC * _NS

  def body(p_ref, c_ref, h_ref, wl_ref, b_ref, wr_ref, o_ref):
    cnt = jnp.sum(c_ref[...], axis=1)[:, None]
    mean = p_ref[...] / jnp.maximum(cnt, 1.0)
    out = (jnp.dot(mean, wl_ref[...], preferred_element_type=jnp.float32)
           + jnp.dot(h_ref[...], wr_ref[...],
                     preferred_element_type=jnp.float32)
           + b_ref[...])
    if relu:
      out = jnp.maximum(out, 0.0)
    o_ref[...] = out

  return pl.pallas_call(
      body,
      grid=(n // blk,),
      in_specs=[
          pl.BlockSpec((blk, d), lambda i: (i, 0)),
          pl.BlockSpec((blk, nw), lambda i: (i, 0)),
          pl.BlockSpec((blk, d), lambda i: (i, 0)),
          pl.BlockSpec((d, d), lambda i: (0, 0)),
          pl.BlockSpec((1, d), lambda i: (0, 0)),
          pl.BlockSpec((d, d), lambda i: (0, 0)),
      ],
      out_specs=pl.BlockSpec((blk, d), lambda i: (i, 0)),
      out_shape=jax.ShapeDtypeStruct((n, d), jnp.float32),
  )


def kernel(x, edge_index, W1l, b1, W1r, W2l, b2, W2r):
  n, d = x.shape
  e = edge_index.shape[1]
  npad = ((n + _NS * 8 - 1) // (_NS * 8)) * (_NS * 8)

  src2 = edge_index[0].reshape(e // _CHUNK, _CHUNK)
  dst2 = edge_index[1].reshape(e // _CHUNK, _CHUNK)
  znd = jnp.zeros((npad, d), jnp.float32)
  b1r = b1.reshape(1, d)
  b2r = b2.reshape(1, d)

  (c1f,) = _sc_count(npad, e)(edge_index[1])
  c1 = c1f.reshape(_NC * _NS, npad).T
  (p1,) = _sc_segment_sum(npad, e, d)(x, src2, dst2, znd)
  h = _tc_combine(n, d, 1000, True)(p1, c1, x, W1l, b1r, W1r)
  (p2,) = _sc_segment_sum(npad, e, d)(h, src2, dst2, znd)
  out = _tc_combine(n, d, 1000, False)(p2, c1, h, W2l, b2r, W2r)
  return out


# speedup vs baseline: 7.9712x; 1.0918x over previous
import functools

import jax
import jax.numpy as jnp
from jax import lax
from jax.experimental import pallas as pl
from jax.experimental.pallas import tpu as pltpu
from jax.experimental.pallas import tpu_sc as plsc

_NC = 1
_NS = 16
_CHUNK = 50
_GPC = 16


@functools.lru_cache(maxsize=None)
def _sc_segment_sum(npad, e, d):
  nw = _NC * _NS
  assert e % (nw * _CHUNK * _GPC) == 0 and npad % (_NS * 8) == 0
  assert _GPC % 8 == 0 and _GPC >= 8
  nchunks = e // (nw * _CHUNK)
  ngen = nchunks // _GPC
  rpt = npad // _NS
  mesh = plsc.VectorSubcoreMesh(core_axis_name="c", subcore_axis_name="s",
                                num_cores=_NC, num_subcores=_NS)

  def body(h_hbm, src_hbm, dst_hbm, znd_hbm, p_out,
           srcv, dstv, b0, b1, b2, b3, g0, g1, g2, g3, s0, s1, s2, s3,
           isem, acc):
    bufs = (b0, b1, b2, b3)
    gsems = (g0, g1, g2, g3)
    ssems = (s0, s1, s2, s3)
    c = lax.axis_index("c")
    s = lax.axis_index("s")
    row0 = (c * _NS + s) * nchunks
    r0 = s * rpt

    pltpu.sync_copy(znd_hbm.at[pl.ds(r0, rpt)], acc.at[pl.ds(r0, rpt)])
    plsc.subcore_barrier()

    def gather(k, kb):
      p, j = (k // _GPC) % 2, k % _GPC
      pltpu.async_copy(h_hbm.at[srcv.at[p, j]], bufs[kb], gsems[kb])

    def wait_gather(k, kb):
      p, j = (k // _GPC) % 2, k % _GPC
      pltpu.make_async_copy(h_hbm.at[srcv.at[p, j]], bufs[kb],
                            gsems[kb]).wait()

    def scat(k, kb):
      p, j = (k // _GPC) % 2, k % _GPC
      pltpu.async_copy(bufs[kb], acc.at[dstv.at[p, j]], ssems[kb],
                       add=True)

    def wait_scat(k, kb):
      p, j = (k // _GPC) % 2, k % _GPC
      pltpu.make_async_copy(bufs[kb], acc.at[dstv.at[p, j]],
                            ssems[kb]).wait()

    def load_idx(g, sync):
      base = row0 + g * _GPC
      p = g % 2
      if sync:
        pltpu.sync_copy(src_hbm.at[pl.ds(base, _GPC)], srcv.at[p])
        pltpu.sync_copy(dst_hbm.at[pl.ds(base, _GPC)], dstv.at[p])
      else:
        pltpu.async_copy(src_hbm.at[pl.ds(base, _GPC)], srcv.at[p], isem)
        pltpu.async_copy(dst_hbm.at[pl.ds(base, _GPC)], dstv.at[p], isem)

    def wait_idx(g):
      base = row0 + g * _GPC
      p = g % 2
      pltpu.make_async_copy(src_hbm.at[pl.ds(base, _GPC)], srcv.at[p],
                            isem).wait()
      pltpu.make_async_copy(dst_hbm.at[pl.ds(base, _GPC)], dstv.at[p],
                            isem).wait()

    load_idx(0, True)
    gather(0, 0)
    gather(1, 1)
    wait_gather(0, 0)
    scat(0, 0)
    gather(2, 2)
    wait_gather(1, 1)
    scat(1, 1)
    gather(3, 3)

    def step(i, cy):
      k0 = 4 * i + 4
      g = k0 // _GPC

      @pl.when(k0 % _GPC == 0)
      def _():
        wait_idx(g)

      @pl.when(jnp.logical_and(k0 % _GPC == 4, g + 1 < ngen))
      def _():
        load_idx(g + 1, False)

      for j in range(4):
        k = k0 + j
        wait_scat(k - 4, j)
        gather(k, j)
        wait_gather(k - 2, (j + 2) % 4)
        scat(k - 2, (j + 2) % 4)
      return cy

    lax.fori_loop(0, (nchunks - 4) // 4, step, 0)
    e1, e2 = nchunks - 2, nchunks - 1
    wait_gather(e1, e1 % 4)
    scat(e1, e1 % 4)
    wait_gather(e2, e2 % 4)
    scat(e2, e2 % 4)
    for j in range(4):
      wait_scat(nchunks - 4 + j, (nchunks - 4 + j) % 4)

    plsc.subcore_barrier()
    pltpu.sync_copy(acc.at[pl.ds(r0, rpt)], p_out.at[pl.ds(r0, rpt)])

  return pl.kernel(
      body,
      out_type=[jax.ShapeDtypeStruct((npad, d), jnp.float32)],
      mesh=mesh,
      scratch_types=[
          pltpu.VMEM((2, _GPC, _CHUNK), jnp.int32),
          pltpu.VMEM((2, _GPC, _CHUNK), jnp.int32),
          pltpu.VMEM((_CHUNK, d), jnp.float32),
          pltpu.VMEM((_CHUNK, d), jnp.float32),
          pltpu.VMEM((_CHUNK, d), jnp.float32),
          pltpu.VMEM((_CHUNK, d), jnp.float32),
          pltpu.SemaphoreType.DMA,
          pltpu.SemaphoreType.DMA,
          pltpu.SemaphoreType.DMA,
          pltpu.SemaphoreType.DMA,
          pltpu.SemaphoreType.DMA,
          pltpu.SemaphoreType.DMA,
          pltpu.SemaphoreType.DMA,
          pltpu.SemaphoreType.DMA,
          pltpu.SemaphoreType.DMA,
          pltpu.VMEM_SHARED((npad, d), jnp.float32),
      ])


@functools.lru_cache(maxsize=None)
def _sc_count(npad, e):
  nw = _NC * _NS
  epw = e // nw
  assert e % (nw * 16) == 0 and npad % 8 == 0 and epw % 8 == 0
  mesh = plsc.VectorSubcoreMesh(core_axis_name="c", subcore_axis_name="s",
                                num_cores=_NC, num_subcores=_NS)

  def body(dst_hbm, cnt_out, dstv, tab):
    c = lax.axis_index("c")
    s = lax.axis_index("s")
    w = c * _NS + s

    pltpu.sync_copy(dst_hbm.at[pl.ds(w * epw, epw)], dstv)
    zeros = jnp.zeros((16,), jnp.float32)

    def zstep(i, cy):
      tab[pl.ds(i * 16, 16)] = zeros
      return cy

    lax.fori_loop(0, npad // 16, zstep, 0)
    ones = jnp.ones((16,), jnp.float32)

    def step(i, cy):
      v = dstv[pl.ds(i * 16, 16)]
      plsc.addupdate_scatter(tab, [v], ones)
      return cy

    lax.fori_loop(0, epw // 16, step, 0)
    pltpu.sync_copy(tab, cnt_out.at[pl.ds(w * npad, npad)])

  return pl.kernel(
      body,
      out_type=[jax.ShapeDtypeStruct((nw * npad,), jnp.float32)],
      mesh=mesh,
      compiler_params=pltpu.CompilerParams(needs_layout_passes=False),
      scratch_types=[
          pltpu.VMEM((epw,), jnp.int32),
          pltpu.VMEM((npad,), jnp.float32),
      ])


@functools.lru_cache(maxsize=None)
def _tc_combine(n, d, blk, relu):
  assert n % blk == 0
  nw = _NC * _NS

  def body(p_ref, c_ref, h_ref, wl_ref, b_ref, wr_ref, o_ref):
    cnt = jnp.sum(c_ref[...], axis=1)[:, None]
    mean = p_ref[...] / jnp.maximum(cnt, 1.0)
    out = (jnp.dot(mean, wl_ref[...], preferred_element_type=jnp.float32)
           + jnp.dot(h_ref[...], wr_ref[...],
                     preferred_element_type=jnp.float32)
           + b_ref[...])
    if relu:
      out = jnp.maximum(out, 0.0)
    o_ref[...] = out

  return pl.pallas_call(
      body,
      grid=(n // blk,),
      in_specs=[
          pl.BlockSpec((blk, d), lambda i: (i, 0)),
          pl.BlockSpec((blk, nw), lambda i: (i, 0)),
          pl.BlockSpec((blk, d), lambda i: (i, 0)),
          pl.BlockSpec((d, d), lambda i: (0, 0)),
          pl.BlockSpec((1, d), lambda i: (0, 0)),
          pl.BlockSpec((d, d), lambda i: (0, 0)),
      ],
      out_specs=pl.BlockSpec((blk, d), lambda i: (i, 0)),
      out_shape=jax.ShapeDtypeStruct((n, d), jnp.float32),
  )


def kernel(x, edge_index, W1l, b1, W1r, W2l, b2, W2r):
  n, d = x.shape
  e = edge_index.shape[1]
  npad = ((n + _NS * 8 - 1) // (_NS * 8)) * (_NS * 8)

  src2 = edge_index[0].reshape(e // _CHUNK, _CHUNK)
  dst2 = edge_index[1].reshape(e // _CHUNK, _CHUNK)
  znd = jnp.zeros((npad, d), jnp.float32)
  b1r = b1.reshape(1, d)
  b2r = b2.reshape(1, d)

  (c1f,) = _sc_count(npad, e)(edge_index[1])
  c1 = c1f.reshape(_NC * _NS, npad).T
  (p1,) = _sc_segment_sum(npad, e, d)(x, src2, dst2, znd)
  h = _tc_combine(n, d, 1000, True)(p1, c1, x, W1l, b1r, W1r)
  (p2,) = _sc_segment_sum(npad, e, d)(h, src2, dst2, znd)
  out = _tc_combine(n, d, 1000, False)(p2, c1, h, W2l, b2r, W2r)
  return out


# speedup vs baseline: 9.0405x; 1.1341x over previous
import functools

import jax
import jax.numpy as jnp
from jax import lax
from jax.experimental import pallas as pl
from jax.experimental.pallas import tpu as pltpu
from jax.experimental.pallas import tpu_sc as plsc

_NC = 1
_NS = 16
_CHUNK = 50
_GPC = 16


@functools.lru_cache(maxsize=None)
def _sc_segment_sum(npad, e, d, nc, gpc):
  dcol = d // nc
  assert e % (_NS * _CHUNK * gpc) == 0 and npad % (_NS * 8) == 0
  assert gpc % 8 == 0 and gpc >= 8
  nchunks = e // (_NS * _CHUNK)
  ngen = nchunks // gpc
  rpt = npad // _NS
  mesh = plsc.VectorSubcoreMesh(core_axis_name="c", subcore_axis_name="s",
                                num_cores=nc, num_subcores=_NS)

  def body(h_hbm, src_hbm, dst_hbm, znd_hbm, p_out,
           srcv, dstv, b0, b1, b2, b3, g0, g1, g2, g3, s0, s1, s2, s3,
           isem, acc):
    bufs = (b0, b1, b2, b3)
    gsems = (g0, g1, g2, g3)
    ssems = (s0, s1, s2, s3)
    c = lax.axis_index("c")
    s = lax.axis_index("s")
    row0 = s * nchunks
    r0 = s * rpt
    hsrc = h_hbm if nc == 1 else h_hbm.at[c]
    pdst = p_out if nc == 1 else p_out.at[c]

    pltpu.sync_copy(znd_hbm.at[pl.ds(r0, rpt)], acc.at[pl.ds(r0, rpt)])
    plsc.subcore_barrier()

    def gather(k, kb):
      p, j = (k // gpc) % 2, k % gpc
      pltpu.async_copy(hsrc.at[srcv.at[p, j]], bufs[kb], gsems[kb])

    def wait_gather(k, kb):
      p, j = (k // gpc) % 2, k % gpc
      pltpu.make_async_copy(hsrc.at[srcv.at[p, j]], bufs[kb],
                            gsems[kb]).wait()

    def scat(k, kb):
      p, j = (k // gpc) % 2, k % gpc
      pltpu.async_copy(bufs[kb], acc.at[dstv.at[p, j]], ssems[kb],
                       add=True)

    def wait_scat(k, kb):
      p, j = (k // gpc) % 2, k % gpc
      pltpu.make_async_copy(bufs[kb], acc.at[dstv.at[p, j]],
                            ssems[kb]).wait()

    def load_idx(g, sync):
      base = row0 + g * gpc
      p = g % 2
      if sync:
        pltpu.sync_copy(src_hbm.at[pl.ds(base, gpc)], srcv.at[p])
        pltpu.sync_copy(dst_hbm.at[pl.ds(base, gpc)], dstv.at[p])
      else:
        pltpu.async_copy(src_hbm.at[pl.ds(base, gpc)], srcv.at[p], isem)
        pltpu.async_copy(dst_hbm.at[pl.ds(base, gpc)], dstv.at[p], isem)

    def wait_idx(g):
      base = row0 + g * gpc
      p = g % 2
      pltpu.make_async_copy(src_hbm.at[pl.ds(base, gpc)], srcv.at[p],
                            isem).wait()
      pltpu.make_async_copy(dst_hbm.at[pl.ds(base, gpc)], dstv.at[p],
                            isem).wait()

    load_idx(0, True)
    gather(0, 0)
    gather(1, 1)
    wait_gather(0, 0)
    scat(0, 0)
    gather(2, 2)
    wait_gather(1, 1)
    scat(1, 1)
    gather(3, 3)

    def step(i, cy):
      k0 = 4 * i + 4
      g = k0 // gpc

      @pl.when(k0 % gpc == 0)
      def _():
        wait_idx(g)

      @pl.when(jnp.logical_and(k0 % gpc == 4, g + 1 < ngen))
      def _():
        load_idx(g + 1, False)

      for j in range(4):
        k = k0 + j
        wait_scat(k - 4, j)
        gather(k, j)
        wait_gather(k - 2, (j + 2) % 4)
        scat(k - 2, (j + 2) % 4)
      return cy

    lax.fori_loop(0, (nchunks - 4) // 4, step, 0)
    e1, e2 = nchunks - 2, nchunks - 1
    wait_gather(e1, e1 % 4)
    scat(e1, e1 % 4)
    wait_gather(e2, e2 % 4)
    scat(e2, e2 % 4)
    for j in range(4):
      wait_scat(nchunks - 4 + j, (nchunks - 4 + j) % 4)

    plsc.subcore_barrier()
    pltpu.sync_copy(acc.at[pl.ds(r0, rpt)], pdst.at[pl.ds(r0, rpt)])

  oshape = (npad, d) if nc == 1 else (nc, npad, dcol)
  return pl.kernel(
      body,
      out_type=[jax.ShapeDtypeStruct(oshape, jnp.float32)],
      mesh=mesh,
      compiler_params=pltpu.CompilerParams(use_tc_tiling_on_sc=False),
      scratch_types=[
          pltpu.VMEM((2, gpc, _CHUNK), jnp.int32),
          pltpu.VMEM((2, gpc, _CHUNK), jnp.int32),
          pltpu.VMEM((_CHUNK, dcol), jnp.float32),
          pltpu.VMEM((_CHUNK, dcol), jnp.float32),
          pltpu.VMEM((_CHUNK, dcol), jnp.float32),
          pltpu.VMEM((_CHUNK, dcol), jnp.float32),
          pltpu.SemaphoreType.DMA,
          pltpu.SemaphoreType.DMA,
          pltpu.SemaphoreType.DMA,
          pltpu.SemaphoreType.DMA,
          pltpu.SemaphoreType.DMA,
          pltpu.SemaphoreType.DMA,
          pltpu.SemaphoreType.DMA,
          pltpu.SemaphoreType.DMA,
          pltpu.SemaphoreType.DMA,
          pltpu.VMEM_SHARED((npad, dcol), jnp.float32),
      ])


@functools.lru_cache(maxsize=None)
def _sc_count(npad, e):
  nw = _NC * _NS
  epw = e // nw
  assert e % (nw * 16) == 0 and npad % 8 == 0 and epw % 8 == 0
  mesh = plsc.VectorSubcoreMesh(core_axis_name="c", subcore_axis_name="s",
                                num_cores=_NC, num_subcores=_NS)

  def body(dst_hbm, cnt_out, dstv, tab):
    c = lax.axis_index("c")
    s = lax.axis_index("s")
    w = c * _NS + s

    pltpu.sync_copy(dst_hbm.at[pl.ds(w * epw, epw)], dstv)
    zeros = jnp.zeros((16,), jnp.float32)

    def zstep(i, cy):
      tab[pl.ds(i * 16, 16)] = zeros
      return cy

    lax.fori_loop(0, npad // 16, zstep, 0)
    ones = jnp.ones((16,), jnp.float32)

    def step(i, cy):
      v = dstv[pl.ds(i * 16, 16)]
      plsc.addupdate_scatter(tab, [v], ones)
      return cy

    lax.fori_loop(0, epw // 16, step, 0)
    pltpu.sync_copy(tab, cnt_out.at[pl.ds(w * npad, npad)])

  return pl.kernel(
      body,
      out_type=[jax.ShapeDtypeStruct((nw * npad,), jnp.float32)],
      mesh=mesh,
      compiler_params=pltpu.CompilerParams(needs_layout_passes=False),
      scratch_types=[
          pltpu.VMEM((epw,), jnp.int32),
          pltpu.VMEM((npad,), jnp.float32),
      ])


@functools.lru_cache(maxsize=None)
def _tc_combine(n, d, blk, relu, split):
  assert n % blk == 0
  nw = _NC * _NS

  def body(p_ref, c_ref, h_ref, wl_ref, b_ref, wr_ref, o_ref):
    cnt = jnp.sum(c_ref[...], axis=1)[:, None]
    if split:
      psum = jnp.concatenate((p_ref[0], p_ref[1]), axis=1)
    else:
      psum = p_ref[...]
    mean = psum / jnp.maximum(cnt, 1.0)
    out = (jnp.dot(mean, wl_ref[...], preferred_element_type=jnp.float32)
           + jnp.dot(h_ref[...], wr_ref[...],
                     preferred_element_type=jnp.float32)
           + b_ref[...])
    if relu:
      out = jnp.maximum(out, 0.0)
    o_ref[...] = out

  return pl.pallas_call(
      body,
      grid=(n // blk,),
      in_specs=[
          (pl.BlockSpec((2, blk, d // 2), lambda i: (0, i, 0)) if split
           else pl.BlockSpec((blk, d), lambda i: (i, 0))),
          pl.BlockSpec((blk, nw), lambda i: (i, 0)),
          pl.BlockSpec((blk, d), lambda i: (i, 0)),
          pl.BlockSpec((d, d), lambda i: (0, 0)),
          pl.BlockSpec((1, d), lambda i: (0, 0)),
          pl.BlockSpec((d, d), lambda i: (0, 0)),
      ],
      out_specs=pl.BlockSpec((blk, d), lambda i: (i, 0)),
      out_shape=jax.ShapeDtypeStruct((n, d), jnp.float32),
  )


def kernel(x, edge_index, W1l, b1, W1r, W2l, b2, W2r):
  n, d = x.shape
  e = edge_index.shape[1]
  npad = ((n + _NS * 8 - 1) // (_NS * 8)) * (_NS * 8)

  src2 = edge_index[0].reshape(e // _CHUNK, _CHUNK)
  dst2 = edge_index[1].reshape(e // _CHUNK, _CHUNK)
  half = d // 2
  znd = jnp.zeros((npad, half), jnp.float32)
  b1r = b1.reshape(1, d)
  b2r = b2.reshape(1, d)

  (c1f,) = _sc_count(npad, e)(edge_index[1])
  c1 = c1f.reshape(_NS, npad).T
  agg = _sc_segment_sum(npad, e, d, 2, 8)

  xs = jnp.stack((x[:, :half], x[:, half:]))
  (p1,) = agg(xs, src2, dst2, znd)
  h = _tc_combine(n, d, 1000, True, True)(p1, c1, x, W1l, b1r, W1r)
  hs = jnp.stack((h[:, :half], h[:, half:]))
  (p2,) = agg(hs, src2, dst2, znd)
  out = _tc_combine(n, d, 1000, False, True)(p2, c1, h, W2l, b2r, W2r)
  return out


# speedup vs baseline: 10.2969x; 1.1390x over previous
import functools

import jax
import jax.numpy as jnp
from jax import lax
from jax.experimental import pallas as pl
from jax.experimental.pallas import tpu as pltpu
from jax.experimental.pallas import tpu_sc as plsc

_NC = 1
_NS = 16
_CHUNK = 100
_GPC = 16


@functools.lru_cache(maxsize=None)
def _sc_segment_sum(npad, e, d, nc, gpc):
  dcol = d // nc
  assert e % (_NS * _CHUNK * gpc) == 0 and npad % (_NS * 8) == 0
  assert gpc % 8 == 0 and gpc >= 8
  nchunks = e // (_NS * _CHUNK)
  ngen = nchunks // gpc
  rpt = npad // _NS
  mesh = plsc.VectorSubcoreMesh(core_axis_name="c", subcore_axis_name="s",
                                num_cores=nc, num_subcores=_NS)

  def body(h_hbm, src_hbm, dst_hbm, znd_hbm, p_out,
           srcv, dstv, b0, b1, b2, b3, g0, g1, g2, g3, s0, s1, s2, s3,
           isem, acc):
    bufs = (b0, b1, b2, b3)
    gsems = (g0, g1, g2, g3)
    ssems = (s0, s1, s2, s3)
    c = lax.axis_index("c")
    s = lax.axis_index("s")
    row0 = s * nchunks
    r0 = s * rpt
    hsrc = h_hbm if nc == 1 else h_hbm.at[c]
    pdst = p_out if nc == 1 else p_out.at[c]

    pltpu.sync_copy(znd_hbm.at[pl.ds(r0, rpt)], acc.at[pl.ds(r0, rpt)])
    plsc.subcore_barrier()

    def gather(k, kb):
      p, j = (k // gpc) % 2, k % gpc
      pltpu.async_copy(hsrc.at[srcv.at[p, j]], bufs[kb], gsems[kb])

    def wait_gather(k, kb):
      p, j = (k // gpc) % 2, k % gpc
      pltpu.make_async_copy(hsrc.at[srcv.at[p, j]], bufs[kb],
                            gsems[kb]).wait()

    def scat(k, kb):
      p, j = (k // gpc) % 2, k % gpc
      pltpu.async_copy(bufs[kb], acc.at[dstv.at[p, j]], ssems[kb],
                       add=True)

    def wait_scat(k, kb):
      p, j = (k // gpc) % 2, k % gpc
      pltpu.make_async_copy(bufs[kb], acc.at[dstv.at[p, j]],
                            ssems[kb]).wait()

    def load_idx(g, sync):
      base = row0 + g * gpc
      p = g % 2
      if sync:
        pltpu.sync_copy(src_hbm.at[pl.ds(base, gpc)], srcv.at[p])
        pltpu.sync_copy(dst_hbm.at[pl.ds(base, gpc)], dstv.at[p])
      else:
        pltpu.async_copy(src_hbm.at[pl.ds(base, gpc)], srcv.at[p], isem)
        pltpu.async_copy(dst_hbm.at[pl.ds(base, gpc)], dstv.at[p], isem)

    def wait_idx(g):
      base = row0 + g * gpc
      p = g % 2
      pltpu.make_async_copy(src_hbm.at[pl.ds(base, gpc)], srcv.at[p],
                            isem).wait()
      pltpu.make_async_copy(dst_hbm.at[pl.ds(base, gpc)], dstv.at[p],
                            isem).wait()

    load_idx(0, True)
    gather(0, 0)
    gather(1, 1)
    wait_gather(0, 0)
    scat(0, 0)
    gather(2, 2)
    wait_gather(1, 1)
    scat(1, 1)
    gather(3, 3)

    def step(i, cy):
      k0 = 4 * i + 4
      g = k0 // gpc

      @pl.when(k0 % gpc == 0)
      def _():
        wait_idx(g)

      @pl.when(jnp.logical_and(k0 % gpc == 4, g + 1 < ngen))
      def _():
        load_idx(g + 1, False)

      for j in range(4):
        k = k0 + j
        wait_scat(k - 4, j)
        gather(k, j)
        wait_gather(k - 2, (j + 2) % 4)
        scat(k - 2, (j + 2) % 4)
      return cy

    lax.fori_loop(0, (nchunks - 4) // 4, step, 0)
    e1, e2 = nchunks - 2, nchunks - 1
    wait_gather(e1, e1 % 4)
    scat(e1, e1 % 4)
    wait_gather(e2, e2 % 4)
    scat(e2, e2 % 4)
    for j in range(4):
      wait_scat(nchunks - 4 + j, (nchunks - 4 + j) % 4)

    plsc.subcore_barrier()
    pltpu.sync_copy(acc.at[pl.ds(r0, rpt)], pdst.at[pl.ds(r0, rpt)])

  oshape = (npad, d) if nc == 1 else (nc, npad, dcol)
  return pl.kernel(
      body,
      out_type=[jax.ShapeDtypeStruct(oshape, jnp.float32)],
      mesh=mesh,
      compiler_params=pltpu.CompilerParams(use_tc_tiling_on_sc=False),
      scratch_types=[
          pltpu.VMEM((2, gpc, _CHUNK), jnp.int32),
          pltpu.VMEM((2, gpc, _CHUNK), jnp.int32),
          pltpu.VMEM((_CHUNK, dcol), jnp.float32),
          pltpu.VMEM((_CHUNK, dcol), jnp.float32),
          pltpu.VMEM((_CHUNK, dcol), jnp.float32),
          pltpu.VMEM((_CHUNK, dcol), jnp.float32),
          pltpu.SemaphoreType.DMA,
          pltpu.SemaphoreType.DMA,
          pltpu.SemaphoreType.DMA,
          pltpu.SemaphoreType.DMA,
          pltpu.SemaphoreType.DMA,
          pltpu.SemaphoreType.DMA,
          pltpu.SemaphoreType.DMA,
          pltpu.SemaphoreType.DMA,
          pltpu.SemaphoreType.DMA,
          pltpu.VMEM_SHARED((npad, dcol), jnp.float32),
      ])


@functools.lru_cache(maxsize=None)
def _sc_count(npad, e):
  nw = _NC * _NS
  epw = e // nw
  assert e % (nw * 16) == 0 and npad % 8 == 0 and epw % 8 == 0
  mesh = plsc.VectorSubcoreMesh(core_axis_name="c", subcore_axis_name="s",
                                num_cores=_NC, num_subcores=_NS)

  def body(dst_hbm, cnt_out, dstv, tab):
    c = lax.axis_index("c")
    s = lax.axis_index("s")
    w = c * _NS + s

    pltpu.sync_copy(dst_hbm.at[pl.ds(w * epw, epw)], dstv)
    zeros = jnp.zeros((16,), jnp.float32)

    def zstep(i, cy):
      tab[pl.ds(i * 16, 16)] = zeros
      return cy

    lax.fori_loop(0, npad // 16, zstep, 0)
    ones = jnp.ones((16,), jnp.float32)

    def step(i, cy):
      v = dstv[pl.ds(i * 16, 16)]
      plsc.addupdate_scatter(tab, [v], ones)
      return cy

    lax.fori_loop(0, epw // 16, step, 0)
    pltpu.sync_copy(tab, cnt_out.at[pl.ds(w * npad, npad)])

  return pl.kernel(
      body,
      out_type=[jax.ShapeDtypeStruct((nw * npad,), jnp.float32)],
      mesh=mesh,
      compiler_params=pltpu.CompilerParams(needs_layout_passes=False),
      scratch_types=[
          pltpu.VMEM((epw,), jnp.int32),
          pltpu.VMEM((npad,), jnp.float32),
      ])


@functools.lru_cache(maxsize=None)
def _tc_combine(n, d, blk, relu, split):
  assert n % blk == 0
  nw = _NC * _NS

  def body(p_ref, c_ref, h_ref, wl_ref, b_ref, wr_ref, o_ref):
    cnt = jnp.sum(c_ref[...], axis=1)[:, None]
    if split:
      psum = jnp.concatenate((p_ref[0], p_ref[1]), axis=1)
    else:
      psum = p_ref[...]
    mean = psum / jnp.maximum(cnt, 1.0)
    out = (jnp.dot(mean, wl_ref[...], preferred_element_type=jnp.float32)
           + jnp.dot(h_ref[...], wr_ref[...],
                     preferred_element_type=jnp.float32)
           + b_ref[...])
    if relu:
      out = jnp.maximum(out, 0.0)
    o_ref[...] = out

  return pl.pallas_call(
      body,
      grid=(n // blk,),
      in_specs=[
          (pl.BlockSpec((2, blk, d // 2), lambda i: (0, i, 0)) if split
           else pl.BlockSpec((blk, d), lambda i: (i, 0))),
          pl.BlockSpec((blk, nw), lambda i: (i, 0)),
          pl.BlockSpec((blk, d), lambda i: (i, 0)),
          pl.BlockSpec((d, d), lambda i: (0, 0)),
          pl.BlockSpec((1, d), lambda i: (0, 0)),
          pl.BlockSpec((d, d), lambda i: (0, 0)),
      ],
      out_specs=pl.BlockSpec((blk, d), lambda i: (i, 0)),
      out_shape=jax.ShapeDtypeStruct((n, d), jnp.float32),
  )


def kernel(x, edge_index, W1l, b1, W1r, W2l, b2, W2r):
  n, d = x.shape
  e = edge_index.shape[1]
  npad = ((n + _NS * 8 - 1) // (_NS * 8)) * (_NS * 8)

  src2 = edge_index[0].reshape(e // _CHUNK, _CHUNK)
  dst2 = edge_index[1].reshape(e // _CHUNK, _CHUNK)
  half = d // 2
  znd = jnp.zeros((npad, half), jnp.float32)
  b1r = b1.reshape(1, d)
  b2r = b2.reshape(1, d)

  (c1f,) = _sc_count(npad, e)(edge_index[1])
  c1 = c1f.reshape(_NS, npad).T
  agg = _sc_segment_sum(npad, e, d, 2, 8)

  xs = jnp.stack((x[:, :half], x[:, half:]))
  (p1,) = agg(xs, src2, dst2, znd)
  h = _tc_combine(n, d, 1000, True, True)(p1, c1, x, W1l, b1r, W1r)
  hs = jnp.stack((h[:, :half], h[:, half:]))
  (p2,) = agg(hs, src2, dst2, znd)
  out = _tc_combine(n, d, 1000, False, True)(p2, c1, h, W2l, b2r, W2r)
  return out


# speedup vs baseline: 10.4676x; 1.0166x over previous
import functools

import jax
import jax.numpy as jnp
from jax import lax
from jax.experimental import pallas as pl
from jax.experimental.pallas import tpu as pltpu
from jax.experimental.pallas import tpu_sc as plsc

_NC = 1
_NS = 16
_CHUNK = 125
_GPC = 16


@functools.lru_cache(maxsize=None)
def _sc_segment_sum(npad, e, d, nc, gpc):
  dcol = d // nc
  assert e % (_NS * _CHUNK * gpc) == 0 and npad % (_NS * 8) == 0
  assert gpc % 8 == 0 and gpc >= 8
  nchunks = e // (_NS * _CHUNK)
  ngen = nchunks // gpc
  rpt = npad // _NS
  mesh = plsc.VectorSubcoreMesh(core_axis_name="c", subcore_axis_name="s",
                                num_cores=nc, num_subcores=_NS)

  def body(h_hbm, src_hbm, dst_hbm, znd_hbm, p_out,
           srcv, dstv, b0, b1, b2, b3, g0, g1, g2, g3, s0, s1, s2, s3,
           isem, acc):
    bufs = (b0, b1, b2, b3)
    gsems = (g0, g1, g2, g3)
    ssems = (s0, s1, s2, s3)
    c = lax.axis_index("c")
    s = lax.axis_index("s")
    row0 = s * nchunks
    r0 = s * rpt
    hsrc = h_hbm if nc == 1 else h_hbm.at[c]
    pdst = p_out if nc == 1 else p_out.at[c]

    pltpu.sync_copy(znd_hbm.at[pl.ds(r0, rpt)], acc.at[pl.ds(r0, rpt)])
    plsc.subcore_barrier()

    def gather(k, kb):
      p, j = (k // gpc) % 2, k % gpc
      pltpu.async_copy(hsrc.at[srcv.at[p, j]], bufs[kb], gsems[kb])

    def wait_gather(k, kb):
      p, j = (k // gpc) % 2, k % gpc
      pltpu.make_async_copy(hsrc.at[srcv.at[p, j]], bufs[kb],
                            gsems[kb]).wait()

    def scat(k, kb):
      p, j = (k // gpc) % 2, k % gpc
      pltpu.async_copy(bufs[kb], acc.at[dstv.at[p, j]], ssems[kb],
                       add=True)

    def wait_scat(k, kb):
      p, j = (k // gpc) % 2, k % gpc
      pltpu.make_async_copy(bufs[kb], acc.at[dstv.at[p, j]],
                            ssems[kb]).wait()

    def load_idx(g, sync):
      base = row0 + g * gpc
      p = g % 2
      if sync:
        pltpu.sync_copy(src_hbm.at[pl.ds(base, gpc)], srcv.at[p])
        pltpu.sync_copy(dst_hbm.at[pl.ds(base, gpc)], dstv.at[p])
      else:
        pltpu.async_copy(src_hbm.at[pl.ds(base, gpc)], srcv.at[p], isem)
        pltpu.async_copy(dst_hbm.at[pl.ds(base, gpc)], dstv.at[p], isem)

    def wait_idx(g):
      base = row0 + g * gpc
      p = g % 2
      pltpu.make_async_copy(src_hbm.at[pl.ds(base, gpc)], srcv.at[p],
                            isem).wait()
      pltpu.make_async_copy(dst_hbm.at[pl.ds(base, gpc)], dstv.at[p],
                            isem).wait()

    load_idx(0, True)
    gather(0, 0)
    gather(1, 1)
    wait_gather(0, 0)
    scat(0, 0)
    gather(2, 2)
    wait_gather(1, 1)
    scat(1, 1)
    gather(3, 3)

    def step(i, cy):
      k0 = 4 * i + 4
      g = k0 // gpc

      @pl.when(k0 % gpc == 0)
      def _():
        wait_idx(g)

      @pl.when(jnp.logical_and(k0 % gpc == 4, g + 1 < ngen))
      def _():
        load_idx(g + 1, False)

      for j in range(4):
        k = k0 + j
        wait_scat(k - 4, j)
        gather(k, j)
        wait_gather(k - 2, (j + 2) % 4)
        scat(k - 2, (j + 2) % 4)
      return cy

    lax.fori_loop(0, (nchunks - 4) // 4, step, 0)
    e1, e2 = nchunks - 2, nchunks - 1
    wait_gather(e1, e1 % 4)
    scat(e1, e1 % 4)
    wait_gather(e2, e2 % 4)
    scat(e2, e2 % 4)
    for j in range(4):
      wait_scat(nchunks - 4 + j, (nchunks - 4 + j) % 4)

    plsc.subcore_barrier()
    pltpu.sync_copy(acc.at[pl.ds(r0, rpt)], pdst.at[pl.ds(r0, rpt)])

  oshape = (npad, d) if nc == 1 else (nc, npad, dcol)
  return pl.kernel(
      body,
      out_type=[jax.ShapeDtypeStruct(oshape, jnp.float32)],
      mesh=mesh,
      compiler_params=pltpu.CompilerParams(use_tc_tiling_on_sc=False),
      scratch_types=[
          pltpu.VMEM((2, gpc, _CHUNK), jnp.int32),
          pltpu.VMEM((2, gpc, _CHUNK), jnp.int32),
          pltpu.VMEM((_CHUNK, dcol), jnp.float32),
          pltpu.VMEM((_CHUNK, dcol), jnp.float32),
          pltpu.VMEM((_CHUNK, dcol), jnp.float32),
          pltpu.VMEM((_CHUNK, dcol), jnp.float32),
          pltpu.SemaphoreType.DMA,
          pltpu.SemaphoreType.DMA,
          pltpu.SemaphoreType.DMA,
          pltpu.SemaphoreType.DMA,
          pltpu.SemaphoreType.DMA,
          pltpu.SemaphoreType.DMA,
          pltpu.SemaphoreType.DMA,
          pltpu.SemaphoreType.DMA,
          pltpu.SemaphoreType.DMA,
          pltpu.VMEM_SHARED((npad, dcol), jnp.float32),
      ])


@functools.lru_cache(maxsize=None)
def _sc_count(npad, e):
  nw = _NC * _NS
  epw = e // nw
  assert e % (nw * 16) == 0 and npad % 8 == 0 and epw % 8 == 0
  mesh = plsc.VectorSubcoreMesh(core_axis_name="c", subcore_axis_name="s",
                                num_cores=_NC, num_subcores=_NS)

  def body(dst_hbm, cnt_out, dstv, tab):
    c = lax.axis_index("c")
    s = lax.axis_index("s")
    w = c * _NS + s

    pltpu.sync_copy(dst_hbm.at[pl.ds(w * epw, epw)], dstv)
    zeros = jnp.zeros((16,), jnp.float32)

    def zstep(i, cy):
      tab[pl.ds(i * 16, 16)] = zeros
      return cy

    lax.fori_loop(0, npad // 16, zstep, 0)
    ones = jnp.ones((16,), jnp.float32)

    def step(i, cy):
      v = dstv[pl.ds(i * 16, 16)]
      plsc.addupdate_scatter(tab, [v], ones)
      return cy

    lax.fori_loop(0, epw // 16, step, 0)
    pltpu.sync_copy(tab, cnt_out.at[pl.ds(w * npad, npad)])

  return pl.kernel(
      body,
      out_type=[jax.ShapeDtypeStruct((nw * npad,), jnp.float32)],
      mesh=mesh,
      compiler_params=pltpu.CompilerParams(needs_layout_passes=False),
      scratch_types=[
          pltpu.VMEM((epw,), jnp.int32),
          pltpu.VMEM((npad,), jnp.float32),
      ])


@functools.lru_cache(maxsize=None)
def _tc_combine(n, d, blk, relu, split):
  assert n % blk == 0
  nw = _NC * _NS

  def body(p_ref, c_ref, h_ref, wl_ref, b_ref, wr_ref, o_ref):
    cnt = jnp.sum(c_ref[...], axis=1)[:, None]
    if split:
      psum = jnp.concatenate((p_ref[0], p_ref[1]), axis=1)
    else:
      psum = p_ref[...]
    mean = psum / jnp.maximum(cnt, 1.0)
    out = (jnp.dot(mean, wl_ref[...], preferred_element_type=jnp.float32)
           + jnp.dot(h_ref[...], wr_ref[...],
                     preferred_element_type=jnp.float32)
           + b_ref[...])
    if relu:
      out = jnp.maximum(out, 0.0)
    o_ref[...] = out

  return pl.pallas_call(
      body,
      grid=(n // blk,),
      in_specs=[
          (pl.BlockSpec((2, blk, d // 2), lambda i: (0, i, 0)) if split
           else pl.BlockSpec((blk, d), lambda i: (i, 0))),
          pl.BlockSpec((blk, nw), lambda i: (i, 0)),
          pl.BlockSpec((blk, d), lambda i: (i, 0)),
          pl.BlockSpec((d, d), lambda i: (0, 0)),
          pl.BlockSpec((1, d), lambda i: (0, 0)),
          pl.BlockSpec((d, d), lambda i: (0, 0)),
      ],
      out_specs=pl.BlockSpec((blk, d), lambda i: (i, 0)),
      out_shape=jax.ShapeDtypeStruct((n, d), jnp.float32),
  )


def kernel(x, edge_index, W1l, b1, W1r, W2l, b2, W2r):
  n, d = x.shape
  e = edge_index.shape[1]
  npad = ((n + _NS * 8 - 1) // (_NS * 8)) * (_NS * 8)

  src2 = edge_index[0].reshape(e // _CHUNK, _CHUNK)
  dst2 = edge_index[1].reshape(e // _CHUNK, _CHUNK)
  half = d // 2
  znd = jnp.zeros((npad, half), jnp.float32)
  b1r = b1.reshape(1, d)
  b2r = b2.reshape(1, d)

  (c1f,) = _sc_count(npad, e)(edge_index[1])
  c1 = c1f.reshape(_NS, npad).T
  agg = _sc_segment_sum(npad, e, d, 2, 8)

  xs = jnp.stack((x[:, :half], x[:, half:]))
  (p1,) = agg(xs, src2, dst2, znd)
  h = _tc_combine(n, d, 1000, True, True)(p1, c1, x, W1l, b1r, W1r)
  hs = jnp.stack((h[:, :half], h[:, half:]))
  (p2,) = agg(hs, src2, dst2, znd)
  out = _tc_combine(n, d, 1000, False, True)(p2, c1, h, W2l, b2r, W2r)
  return out


# speedup vs baseline: 10.9339x; 1.0446x over previous
import functools

import jax
import jax.numpy as jnp
from jax import lax
from jax.experimental import pallas as pl
from jax.experimental.pallas import tpu as pltpu
from jax.experimental.pallas import tpu_sc as plsc

_NC = 1
_NS = 16
_CHUNK = 125
_GPC = 16


@functools.lru_cache(maxsize=None)
def _sc_segment_sum(npad, e, d, nc, gpc):
  dcol = d // nc
  assert e % (_NS * _CHUNK * gpc) == 0 and npad % (_NS * 8) == 0
  assert gpc % 8 == 0 and gpc >= 8
  nchunks = e // (_NS * _CHUNK)
  ngen = nchunks // gpc
  rpt = npad // _NS
  mesh = plsc.VectorSubcoreMesh(core_axis_name="c", subcore_axis_name="s",
                                num_cores=nc, num_subcores=_NS)

  def body(h_hbm, src_hbm, dst_hbm, znd_hbm, p_out,
           srcv, dstv, b0, b1, b2, b3, g0, g1, g2, g3, s0, s1, s2, s3,
           isem, acc):
    bufs = (b0, b1, b2, b3)
    gsems = (g0, g1, g2, g3)
    ssems = (s0, s1, s2, s3)
    c = lax.axis_index("c")
    s = lax.axis_index("s")
    row0 = s * nchunks
    r0 = s * rpt
    hsrc = h_hbm if nc == 1 else h_hbm.at[c]
    pdst = p_out if nc == 1 else p_out.at[c]

    pltpu.sync_copy(znd_hbm.at[pl.ds(r0, rpt)], acc.at[pl.ds(r0, rpt)])
    plsc.subcore_barrier()

    def gather(k, kb):
      p, j = (k // gpc) % 2, k % gpc
      pltpu.async_copy(hsrc.at[srcv.at[p, j]], bufs[kb], gsems[kb])

    def wait_gather(k, kb):
      p, j = (k // gpc) % 2, k % gpc
      pltpu.make_async_copy(hsrc.at[srcv.at[p, j]], bufs[kb],
                            gsems[kb]).wait()

    def scat(k, kb):
      p, j = (k // gpc) % 2, k % gpc
      pltpu.async_copy(bufs[kb], acc.at[dstv.at[p, j]], ssems[kb],
                       add=True)

    def wait_scat(k, kb):
      p, j = (k // gpc) % 2, k % gpc
      pltpu.make_async_copy(bufs[kb], acc.at[dstv.at[p, j]],
                            ssems[kb]).wait()

    def load_idx(g, sync):
      base = row0 + g * gpc
      p = g % 2
      if sync:
        pltpu.sync_copy(src_hbm.at[pl.ds(base, gpc)], srcv.at[p])
        pltpu.sync_copy(dst_hbm.at[pl.ds(base, gpc)], dstv.at[p])
      else:
        pltpu.async_copy(src_hbm.at[pl.ds(base, gpc)], srcv.at[p], isem)
        pltpu.async_copy(dst_hbm.at[pl.ds(base, gpc)], dstv.at[p], isem)

    def wait_idx(g):
      base = row0 + g * gpc
      p = g % 2
      pltpu.make_async_copy(src_hbm.at[pl.ds(base, gpc)], srcv.at[p],
                            isem).wait()
      pltpu.make_async_copy(dst_hbm.at[pl.ds(base, gpc)], dstv.at[p],
                            isem).wait()

    load_idx(0, True)
    gather(0, 0)
    gather(1, 1)
    wait_gather(0, 0)
    scat(0, 0)
    gather(2, 2)
    wait_gather(1, 1)
    scat(1, 1)
    gather(3, 3)

    def step(i, cy):
      k0 = 4 * i + 4
      g = k0 // gpc

      @pl.when(k0 % gpc == 0)
      def _():
        wait_idx(g)

      @pl.when(jnp.logical_and(k0 % gpc == 4, g + 1 < ngen))
      def _():
        load_idx(g + 1, False)

      for j in range(4):
        k = k0 + j
        wait_scat(k - 4, j)
        gather(k, j)
        wait_gather(k - 2, (j + 2) % 4)
        scat(k - 2, (j + 2) % 4)
      return cy

    lax.fori_loop(0, (nchunks - 4) // 4, step, 0)
    e1, e2 = nchunks - 2, nchunks - 1
    wait_gather(e1, e1 % 4)
    scat(e1, e1 % 4)
    wait_gather(e2, e2 % 4)
    scat(e2, e2 % 4)
    for j in range(4):
      wait_scat(nchunks - 4 + j, (nchunks - 4 + j) % 4)

    plsc.subcore_barrier()
    pltpu.sync_copy(acc.at[pl.ds(r0, rpt)], pdst.at[pl.ds(r0, rpt)])

  oshape = (npad, d) if nc == 1 else (nc, npad, dcol)
  return pl.kernel(
      body,
      out_type=[jax.ShapeDtypeStruct(oshape, jnp.float32)],
      mesh=mesh,
      compiler_params=pltpu.CompilerParams(use_tc_tiling_on_sc=False),
      scratch_types=[
          pltpu.VMEM((2, gpc, _CHUNK), jnp.int32),
          pltpu.VMEM((2, gpc, _CHUNK), jnp.int32),
          pltpu.VMEM((_CHUNK, dcol), jnp.float32),
          pltpu.VMEM((_CHUNK, dcol), jnp.float32),
          pltpu.VMEM((_CHUNK, dcol), jnp.float32),
          pltpu.VMEM((_CHUNK, dcol), jnp.float32),
          pltpu.SemaphoreType.DMA,
          pltpu.SemaphoreType.DMA,
          pltpu.SemaphoreType.DMA,
          pltpu.SemaphoreType.DMA,
          pltpu.SemaphoreType.DMA,
          pltpu.SemaphoreType.DMA,
          pltpu.SemaphoreType.DMA,
          pltpu.SemaphoreType.DMA,
          pltpu.SemaphoreType.DMA,
          pltpu.VMEM_SHARED((npad, dcol), jnp.float32),
      ])


@functools.lru_cache(maxsize=None)
def _sc_count(npad, e):
  nw = _NC * _NS
  epw = e // nw
  assert e % (nw * 16) == 0 and npad % 8 == 0 and epw % 8 == 0
  mesh = plsc.VectorSubcoreMesh(core_axis_name="c", subcore_axis_name="s",
                                num_cores=_NC, num_subcores=_NS)

  def body(dst_hbm, cnt_out, dstv, tab):
    c = lax.axis_index("c")
    s = lax.axis_index("s")
    w = c * _NS + s

    pltpu.sync_copy(dst_hbm.at[pl.ds(w * epw, epw)], dstv)
    zeros = jnp.zeros((16,), jnp.float32)

    def zstep(i, cy):
      tab[pl.ds(i * 16, 16)] = zeros
      return cy

    lax.fori_loop(0, npad // 16, zstep, 0)
    ones = jnp.ones((16,), jnp.float32)

    def step(i, cy):
      v = dstv[pl.ds(i * 16, 16)]
      plsc.addupdate_scatter(tab, [v], ones)
      return cy

    lax.fori_loop(0, epw // 16, step, 0)
    pltpu.sync_copy(tab, cnt_out.at[pl.ds(w * npad, npad)])

  return pl.kernel(
      body,
      out_type=[jax.ShapeDtypeStruct((nw * npad,), jnp.float32)],
      mesh=mesh,
      compiler_params=pltpu.CompilerParams(needs_layout_passes=False),
      scratch_types=[
          pltpu.VMEM((epw,), jnp.int32),
          pltpu.VMEM((npad,), jnp.float32),
      ])


@functools.lru_cache(maxsize=None)
def _tc_combine(n, d, blk, relu, split):
  assert n % blk == 0
  nw = _NC * _NS

  def body(p_ref, c_ref, h_ref, wl_ref, b_ref, wr_ref, o_ref):
    cnt = jnp.sum(c_ref[...], axis=1)[:, None]
    recip = 1.0 / jnp.maximum(cnt, 1.0)
    if split:
      out = (jnp.dot(p_ref[0] * recip, wl_ref[0],
                     preferred_element_type=jnp.float32)
             + jnp.dot(p_ref[1] * recip, wl_ref[1],
                       preferred_element_type=jnp.float32))
    else:
      out = jnp.dot(p_ref[...] * recip, wl_ref[...],
                    preferred_element_type=jnp.float32)
    out = (out + jnp.dot(h_ref[...], wr_ref[...],
                         preferred_element_type=jnp.float32)
           + b_ref[...])
    if relu:
      out = jnp.maximum(out, 0.0)
    o_ref[...] = out

  return pl.pallas_call(
      body,
      grid=(n // blk,),
      in_specs=[
          (pl.BlockSpec((2, blk, d // 2), lambda i: (0, i, 0)) if split
           else pl.BlockSpec((blk, d), lambda i: (i, 0))),
          pl.BlockSpec((blk, nw), lambda i: (i, 0)),
          pl.BlockSpec((blk, d), lambda i: (i, 0)),
          (pl.BlockSpec((2, d // 2, d), lambda i: (0, 0, 0)) if split
           else pl.BlockSpec((d, d), lambda i: (0, 0))),
          pl.BlockSpec((1, d), lambda i: (0, 0)),
          pl.BlockSpec((d, d), lambda i: (0, 0)),
      ],
      out_specs=pl.BlockSpec((blk, d), lambda i: (i, 0)),
      out_shape=jax.ShapeDtypeStruct((n, d), jnp.float32),
  )


def kernel(x, edge_index, W1l, b1, W1r, W2l, b2, W2r):
  n, d = x.shape
  e = edge_index.shape[1]
  npad = ((n + _NS * 8 - 1) // (_NS * 8)) * (_NS * 8)

  src2 = edge_index[0].reshape(e // _CHUNK, _CHUNK)
  dst2 = edge_index[1].reshape(e // _CHUNK, _CHUNK)
  half = d // 2
  znd = jnp.zeros((npad, half), jnp.float32)
  b1r = b1.reshape(1, d)
  b2r = b2.reshape(1, d)

  (c1f,) = _sc_count(npad, e)(edge_index[1])
  c1 = c1f.reshape(_NS, npad).T
  agg = _sc_segment_sum(npad, e, d, 2, 8)
  W1s = W1l.reshape(2, half, d)
  W2s = W2l.reshape(2, half, d)

  xs = jnp.stack((x[:, :half], x[:, half:]))
  (p1,) = agg(xs, src2, dst2, znd)
  h = _tc_combine(n, d, 1000, True, True)(p1, c1, x, W1s, b1r, W1r)
  hs = jnp.stack((h[:, :half], h[:, half:]))
  (p2,) = agg(hs, src2, dst2, znd)
  out = _tc_combine(n, d, 1000, False, True)(p2, c1, h, W2s, b2r, W2r)
  return out


# speedup vs baseline: 11.1394x; 1.0188x over previous
import functools

import jax
import jax.numpy as jnp
from jax import lax
from jax.experimental import pallas as pl
from jax.experimental.pallas import tpu as pltpu
from jax.experimental.pallas import tpu_sc as plsc

_NC = 1
_NS = 16
_CHUNK = 125
_GPC = 16


@functools.lru_cache(maxsize=None)
def _sc_segment_sum(npad, e, d, nc, gpc):
  dcol = d // nc
  assert e % (_NS * _CHUNK * gpc) == 0 and npad % (_NS * 8) == 0
  assert gpc % 8 == 0 and gpc >= 8
  nchunks = e // (_NS * _CHUNK)
  ngen = nchunks // gpc
  rpt = npad // _NS
  mesh = plsc.VectorSubcoreMesh(core_axis_name="c", subcore_axis_name="s",
                                num_cores=nc, num_subcores=_NS)

  def body(h_hbm, src_hbm, dst_hbm, znd_hbm, p_out,
           srcv, dstv, b0, b1, b2, b3, g0, g1, g2, g3, s0, s1, s2, s3,
           isem, acc):
    bufs = (b0, b1, b2, b3)
    gsems = (g0, g1, g2, g3)
    ssems = (s0, s1, s2, s3)
    c = lax.axis_index("c")
    s = lax.axis_index("s")
    row0 = s * nchunks
    r0 = s * rpt
    hsrc = h_hbm if nc == 1 else h_hbm.at[c]
    pdst = p_out if nc == 1 else p_out.at[c]

    pltpu.sync_copy(znd_hbm.at[pl.ds(r0, rpt)], acc.at[pl.ds(r0, rpt)])
    plsc.subcore_barrier()

    def gather(k, kb):
      p, j = (k // gpc) % 2, k % gpc
      pltpu.async_copy(hsrc.at[srcv.at[p, j]], bufs[kb], gsems[kb])

    def wait_gather(k, kb):
      p, j = (k // gpc) % 2, k % gpc
      pltpu.make_async_copy(hsrc.at[srcv.at[p, j]], bufs[kb],
                            gsems[kb]).wait()

    def scat(k, kb):
      p, j = (k // gpc) % 2, k % gpc
      pltpu.async_copy(bufs[kb], acc.at[dstv.at[p, j]], ssems[kb],
                       add=True)

    def wait_scat(k, kb):
      p, j = (k // gpc) % 2, k % gpc
      pltpu.make_async_copy(bufs[kb], acc.at[dstv.at[p, j]],
                            ssems[kb]).wait()

    def load_idx(g, sync):
      base = row0 + g * gpc
      p = g % 2
      if sync:
        pltpu.sync_copy(src_hbm.at[pl.ds(base, gpc)], srcv.at[p])
        pltpu.sync_copy(dst_hbm.at[pl.ds(base, gpc)], dstv.at[p])
      else:
        pltpu.async_copy(src_hbm.at[pl.ds(base, gpc)], srcv.at[p], isem)
        pltpu.async_copy(dst_hbm.at[pl.ds(base, gpc)], dstv.at[p], isem)

    def wait_idx(g):
      base = row0 + g * gpc
      p = g % 2
      pltpu.make_async_copy(src_hbm.at[pl.ds(base, gpc)], srcv.at[p],
                            isem).wait()
      pltpu.make_async_copy(dst_hbm.at[pl.ds(base, gpc)], dstv.at[p],
                            isem).wait()

    load_idx(0, True)
    gather(0, 0)
    gather(1, 1)
    wait_gather(0, 0)
    scat(0, 0)
    gather(2, 2)
    wait_gather(1, 1)
    scat(1, 1)
    gather(3, 3)

    def step(i, cy):
      k0 = 4 * i + 4
      g = k0 // gpc

      @pl.when(k0 % gpc == 0)
      def _():
        wait_idx(g)

      @pl.when(jnp.logical_and(k0 % gpc == 4, g + 1 < ngen))
      def _():
        load_idx(g + 1, False)

      for j in range(4):
        k = k0 + j
        wait_scat(k - 4, j)
        gather(k, j)
        wait_gather(k - 2, (j + 2) % 4)
        scat(k - 2, (j + 2) % 4)
      return cy

    lax.fori_loop(0, (nchunks - 4) // 4, step, 0)
    e1, e2 = nchunks - 2, nchunks - 1
    wait_gather(e1, e1 % 4)
    scat(e1, e1 % 4)
    wait_gather(e2, e2 % 4)
    scat(e2, e2 % 4)
    for j in range(4):
      wait_scat(nchunks - 4 + j, (nchunks - 4 + j) % 4)

    plsc.subcore_barrier()
    pltpu.sync_copy(acc.at[pl.ds(r0, rpt)], pdst.at[pl.ds(r0, rpt)])

  oshape = (npad, d) if nc == 1 else (nc, npad, dcol)
  return pl.kernel(
      body,
      out_type=[jax.ShapeDtypeStruct(oshape, jnp.float32)],
      mesh=mesh,
      compiler_params=pltpu.CompilerParams(use_tc_tiling_on_sc=False),
      scratch_types=[
          pltpu.VMEM((2, gpc, _CHUNK), jnp.int32),
          pltpu.VMEM((2, gpc, _CHUNK), jnp.int32),
          pltpu.VMEM((_CHUNK, dcol), jnp.float32),
          pltpu.VMEM((_CHUNK, dcol), jnp.float32),
          pltpu.VMEM((_CHUNK, dcol), jnp.float32),
          pltpu.VMEM((_CHUNK, dcol), jnp.float32),
          pltpu.SemaphoreType.DMA,
          pltpu.SemaphoreType.DMA,
          pltpu.SemaphoreType.DMA,
          pltpu.SemaphoreType.DMA,
          pltpu.SemaphoreType.DMA,
          pltpu.SemaphoreType.DMA,
          pltpu.SemaphoreType.DMA,
          pltpu.SemaphoreType.DMA,
          pltpu.SemaphoreType.DMA,
          pltpu.VMEM_SHARED((npad, dcol), jnp.float32),
      ])


@functools.lru_cache(maxsize=None)
def _sc_count(npad, e):
  nw = _NC * _NS
  epw = e // nw
  assert e % (nw * 16) == 0 and npad % 8 == 0 and epw % 8 == 0
  mesh = plsc.VectorSubcoreMesh(core_axis_name="c", subcore_axis_name="s",
                                num_cores=_NC, num_subcores=_NS)

  def body(dst_hbm, cnt_out, dstv, tab):
    c = lax.axis_index("c")
    s = lax.axis_index("s")
    w = c * _NS + s

    pltpu.sync_copy(dst_hbm.at[pl.ds(w * epw, epw)], dstv)
    zeros = jnp.zeros((16,), jnp.float32)

    def zstep(i, cy):
      tab[pl.ds(i * 16, 16)] = zeros
      return cy

    lax.fori_loop(0, npad // 16, zstep, 0)
    ones = jnp.ones((16,), jnp.float32)

    def step(i, cy):
      v = dstv[pl.ds(i * 16, 16)]
      plsc.addupdate_scatter(tab, [v], ones)
      return cy

    lax.fori_loop(0, epw // 16, step, 0)
    pltpu.sync_copy(tab, cnt_out.at[pl.ds(w * npad, npad)])

  return pl.kernel(
      body,
      out_type=[jax.ShapeDtypeStruct((nw * npad,), jnp.float32)],
      mesh=mesh,
      compiler_params=pltpu.CompilerParams(needs_layout_passes=False),
      scratch_types=[
          pltpu.VMEM((epw,), jnp.int32),
          pltpu.VMEM((npad,), jnp.float32),
      ])


@functools.lru_cache(maxsize=None)
def _tc_combine(n, d, blk, relu, split):
  assert n % blk == 0
  nw = _NC * _NS

  def body(p_ref, c_ref, h_ref, wl_ref, b_ref, wr_ref, o_ref):
    cnt = jnp.sum(c_ref[...], axis=1)[:, None]
    recip = 1.0 / jnp.maximum(cnt, 1.0)
    if split:
      out = (jnp.dot(p_ref[0] * recip, wl_ref[0],
                     preferred_element_type=jnp.float32)
             + jnp.dot(p_ref[1] * recip, wl_ref[1],
                       preferred_element_type=jnp.float32))
    else:
      out = jnp.dot(p_ref[...] * recip, wl_ref[...],
                    preferred_element_type=jnp.float32)
    out = (out + jnp.dot(h_ref[...], wr_ref[...],
                         preferred_element_type=jnp.float32)
           + b_ref[...])
    if relu:
      out = jnp.maximum(out, 0.0)
    o_ref[...] = out

  return pl.pallas_call(
      body,
      grid=(n // blk,),
      in_specs=[
          (pl.BlockSpec((2, blk, d // 2), lambda i: (0, i, 0)) if split
           else pl.BlockSpec((blk, d), lambda i: (i, 0))),
          pl.BlockSpec((blk, nw), lambda i: (i, 0)),
          pl.BlockSpec((blk, d), lambda i: (i, 0)),
          (pl.BlockSpec((2, d // 2, d), lambda i: (0, 0, 0)) if split
           else pl.BlockSpec((d, d), lambda i: (0, 0))),
          pl.BlockSpec((1, d), lambda i: (0, 0)),
          pl.BlockSpec((d, d), lambda i: (0, 0)),
      ],
      out_specs=pl.BlockSpec((blk, d), lambda i: (i, 0)),
      out_shape=jax.ShapeDtypeStruct((n, d), jnp.float32),
  )


def kernel(x, edge_index, W1l, b1, W1r, W2l, b2, W2r):
  n, d = x.shape
  e = edge_index.shape[1]
  npad = ((n + _NS * 8 - 1) // (_NS * 8)) * (_NS * 8)

  src2 = edge_index[0].reshape(e // _CHUNK, _CHUNK)
  dst2 = edge_index[1].reshape(e // _CHUNK, _CHUNK)
  half = d // 2
  znd = jnp.zeros((npad, half), jnp.float32)
  b1r = b1.reshape(1, d)
  b2r = b2.reshape(1, d)

  (c1f,) = _sc_count(npad, e)(edge_index[1])
  c1 = c1f.reshape(_NS, npad).T
  agg = _sc_segment_sum(npad, e, d, 2, 16)
  W1s = W1l.reshape(2, half, d)
  W2s = W2l.reshape(2, half, d)

  xs = jnp.stack((x[:, :half], x[:, half:]))
  (p1,) = agg(xs, src2, dst2, znd)
  h = _tc_combine(n, d, 2000, True, True)(p1, c1, x, W1s, b1r, W1r)
  hs = jnp.stack((h[:, :half], h[:, half:]))
  (p2,) = agg(hs, src2, dst2, znd)
  out = _tc_combine(n, d, 2000, False, True)(p2, c1, h, W2s, b2r, W2r)
  return out
